# Initial kernel scaffold; baseline (speedup 1.0000x reference)
import functools

import jax
import jax.numpy as jnp
from jax import lax
from jax.experimental import pallas as pl
from jax.experimental.pallas import tpu as pltpu
from jax.experimental.pallas import tpu_sc as plsc

N_NODES = 10000
N_PAD = 10240
N_EDGES = 320000
FEAT = 128
ACC_W = FEAT + 16
NC = 2
NS = 16
NW = NC * NS
EPT = N_EDGES // NW
CHUNK = 80
NCHUNK = EPT // CHUNK
RPT = N_PAD // NS
ZROWS = 80


def _project_body(x_ref, w_ref, a_ref, h_ref, s_ref):
    h = jnp.dot(x_ref[...], w_ref[...], preferred_element_type=jnp.float32)
    h_ref[...] = h
    s_ref[...] = jnp.dot(h, a_ref[...], preferred_element_type=jnp.float32)


def _project(x, W, a2):
    blk = 2000
    return pl.pallas_call(
        _project_body,
        grid=(N_NODES // blk,),
        in_specs=[
            pl.BlockSpec((blk, FEAT), lambda i: (i, 0)),
            pl.BlockSpec((FEAT, FEAT), lambda i: (0, 0)),
            pl.BlockSpec((FEAT, 2), lambda i: (0, 0)),
        ],
        out_specs=[
            pl.BlockSpec((blk, FEAT), lambda i: (i, 0)),
            pl.BlockSpec((blk, 2), lambda i: (i, 0)),
        ],
        out_shape=[
            jax.ShapeDtypeStruct((N_NODES, FEAT), jnp.float32),
            jax.ShapeDtypeStruct((N_NODES, 2), jnp.float32),
        ],
    )(x, W, a2)


def _sc_edge_pass(h, s1, s2, src3, dst3):
    mesh = plsc.VectorSubcoreMesh(core_axis_name="c", subcore_axis_name="s")

    @functools.partial(
        pl.kernel,
        mesh=mesh,
        out_type=jax.ShapeDtypeStruct((NC, N_PAD, ACC_W), jnp.float32),
        scratch_types=[
            pltpu.VMEM((NCHUNK, CHUNK), jnp.int32),
            pltpu.VMEM((NCHUNK, CHUNK), jnp.int32),
            pltpu.VMEM((N_NODES,), jnp.float32),
            pltpu.VMEM((N_NODES,), jnp.float32),
            pltpu.VMEM((CHUNK,), jnp.float32),
            pltpu.VMEM((CHUNK, FEAT), jnp.float32),
            pltpu.VMEM((CHUNK, ACC_W), jnp.float32),
            pltpu.VMEM((ZROWS, ACC_W), jnp.float32),
            pltpu.VMEM_SHARED((N_PAD, ACC_W), jnp.float32),
            pltpu.SemaphoreType.DMA,
        ],
    )
    def k(h_hbm, s1_hbm, s2_hbm, src_hbm, dst_hbm, out_hbm,
          src_v, dst_v, s1_v, s2_v, w_v, rows_v, scaled_v, z_v, acc_sh, sem):
        cid = lax.axis_index("c")
        sid = lax.axis_index("s")
        wid = cid * NS + sid

        zero16 = jnp.zeros((16,), jnp.float32)

        def zrow(r, _):
            for cc in range(ACC_W // 16):
                z_v[r, pl.ds(cc * 16, 16)] = zero16
            return 0

        lax.fori_loop(0, ZROWS, zrow, 0)
        for t in range(RPT // ZROWS):
            pltpu.sync_copy(z_v, acc_sh.at[pl.ds(sid * RPT + t * ZROWS, ZROWS)])

        pltpu.sync_copy(src_hbm.at[wid], src_v)
        pltpu.sync_copy(dst_hbm.at[wid], dst_v)
        pltpu.sync_copy(s1_hbm, s1_v)
        pltpu.sync_copy(s2_hbm, s2_v)

        plsc.subcore_barrier()

        def chunk_body(g, _):
            gather = pltpu.async_copy(h_hbm.at[src_v.at[g]], rows_v, sem)
            for j in range(CHUNK // 16):
                sv = src_v[g, pl.ds(j * 16, 16)]
                dv = dst_v[g, pl.ds(j * 16, 16)]
                e = plsc.load_gather(s1_v, [sv]) + plsc.load_gather(s2_v, [dv])
                e = jnp.where(e > 0.0, e, 0.2 * e)
                w_v[pl.ds(j * 16, 16)] = jnp.exp(e)
            gather.wait()

            def scale_body(r, _):
                wv = plsc.load_gather(w_v, [jnp.full((16,), r, jnp.int32)])
                for cc in range(FEAT // 16):
                    scaled_v[r, pl.ds(cc * 16, 16)] = (
                        rows_v[r, pl.ds(cc * 16, 16)] * wv)
                scaled_v[r, pl.ds(FEAT, 16)] = wv
                return 0

            lax.fori_loop(0, CHUNK, scale_body, 0)
            pltpu.sync_copy(scaled_v, acc_sh.at[dst_v.at[g]], add=True)
            return 0

        lax.fori_loop(0, NCHUNK, chunk_body, 0)

        plsc.subcore_barrier()
        pltpu.sync_copy(acc_sh.at[pl.ds(sid * RPT, RPT)],
                        out_hbm.at[cid, pl.ds(sid * RPT, RPT)])

    return k(h, s1, s2, src3, dst3)


def _finalize_body(p_ref, o_ref):
    t = p_ref[0] + p_ref[1]
    den = jnp.sum(t[:, FEAT:ACC_W], axis=1, keepdims=True) * (1.0 / 16.0)
    den = jnp.where(den == 0.0, 1.0, den)
    x = t[:, :FEAT] / den
    o_ref[...] = jnp.where(x > 0.0, x, jnp.exp(jnp.minimum(x, 0.0)) - 1.0)


def _finalize(partials):
    blk = 2048
    return pl.pallas_call(
        _finalize_body,
        grid=(N_PAD // blk,),
        in_specs=[pl.BlockSpec((NC, blk, ACC_W), lambda i: (0, i, 0))],
        out_specs=pl.BlockSpec((blk, FEAT), lambda i: (i, 0)),
        out_shape=jax.ShapeDtypeStruct((N_PAD, FEAT), jnp.float32),
    )(partials)


def kernel(features, edge_src, edge_dst, W, a):
    x = features.reshape(N_NODES, FEAT)
    a2 = a.reshape(2, FEAT).T
    h, s = _project(x, W, a2)
    s1 = s[:, 0]
    s2 = s[:, 1]
    src3 = edge_src.reshape(NW, NCHUNK, CHUNK)
    dst3 = edge_dst.reshape(NW, NCHUNK, CHUNK)
    partials = _sc_edge_pass(h, s1, s2, src3, dst3)
    out = _finalize(partials)
    return out[:N_NODES].reshape(N_NODES, 1, FEAT)


# speedup vs baseline: 7.1104x; 7.1104x over previous
import functools

import jax
import jax.numpy as jnp
from jax import lax
from jax.experimental import pallas as pl
from jax.experimental.pallas import tpu as pltpu
from jax.experimental.pallas import tpu_sc as plsc

N_NODES = 10000
N_PAD = 10240
N_EDGES = 320000
FEAT = 128
HFEAT = FEAT // 2
NC = 2
NS = 16
EPT = N_EDGES // NS
CHUNK = 80
NCHUNK = EPT // CHUNK
RPT = N_PAD // NS
ZROWS = 80
DROWS = N_PAD // 128
DRPT = DROWS // NS


def _project_body(x_ref, w_ref, a_ref, h_ref, s_ref):
    h = jnp.dot(x_ref[...], w_ref[...], preferred_element_type=jnp.float32)
    h_ref[0] = h[:, :HFEAT]
    h_ref[1] = h[:, HFEAT:]
    s_ref[...] = jnp.dot(h, a_ref[...], preferred_element_type=jnp.float32)


def _project(x, W, a2):
    blk = 2000
    return pl.pallas_call(
        _project_body,
        grid=(N_NODES // blk,),
        in_specs=[
            pl.BlockSpec((blk, FEAT), lambda i: (i, 0)),
            pl.BlockSpec((FEAT, FEAT), lambda i: (0, 0)),
            pl.BlockSpec((FEAT, 2), lambda i: (0, 0)),
        ],
        out_specs=[
            pl.BlockSpec((NC, blk, HFEAT), lambda i: (0, i, 0)),
            pl.BlockSpec((blk, 2), lambda i: (i, 0)),
        ],
        out_shape=[
            jax.ShapeDtypeStruct((NC, N_NODES, HFEAT), jnp.float32),
            jax.ShapeDtypeStruct((N_NODES, 2), jnp.float32),
        ],
    )(x, W, a2)


def _sc_edge_pass(hsplit, s1, s2, src3, dst3):
    mesh = plsc.VectorSubcoreMesh(core_axis_name="c", subcore_axis_name="s")

    @functools.partial(
        pl.kernel,
        mesh=mesh,
        compiler_params=pltpu.CompilerParams(
            needs_layout_passes=False, use_tc_tiling_on_sc=False),
        out_type=[
            jax.ShapeDtypeStruct((NC, N_PAD, HFEAT), jnp.float32),
            jax.ShapeDtypeStruct((NC, DROWS, 128), jnp.float32),
        ],
        scratch_types=[
            pltpu.VMEM((NCHUNK, CHUNK), jnp.int32),
            pltpu.VMEM((NCHUNK, CHUNK), jnp.int32),
            pltpu.VMEM((N_NODES,), jnp.float32),
            pltpu.VMEM((N_NODES,), jnp.float32),
            pltpu.VMEM((CHUNK,), jnp.float32),
            pltpu.VMEM((CHUNK, HFEAT), jnp.float32),
            pltpu.VMEM((CHUNK, HFEAT), jnp.float32),
            pltpu.VMEM((ZROWS, HFEAT), jnp.float32),
            pltpu.VMEM((DROWS, 128), jnp.float32),
            pltpu.VMEM((DROWS,), jnp.int32),
            pltpu.VMEM_SHARED((N_PAD, HFEAT), jnp.float32),
            pltpu.VMEM_SHARED((DROWS, 128), jnp.float32),
            pltpu.SemaphoreType.DMA,
        ],
    )
    def k(h_hbm, s1_hbm, s2_hbm, src_hbm, dst_hbm, out_hbm, den_hbm,
          src_v, dst_v, s1_v, s2_v, w_v, rows_v, scaled_v, z_v,
          den_v, idx_v, acc_sh, den_sh, sem):
        cid = lax.axis_index("c")
        sid = lax.axis_index("s")

        zero16 = jnp.zeros((16,), jnp.float32)
        iota16 = lax.iota(jnp.int32, 16)

        def zrow(r, _):
            for cc in range(HFEAT // 16):
                z_v[r, pl.ds(cc * 16, 16)] = zero16
            return 0

        lax.fori_loop(0, ZROWS, zrow, 0)
        for t in range(RPT // ZROWS):
            pltpu.sync_copy(z_v, acc_sh.at[pl.ds(sid * RPT + t * ZROWS, ZROWS)])

        def zden(r, _):
            for cc in range(128 // 16):
                den_v[r, pl.ds(cc * 16, 16)] = zero16
            return 0

        lax.fori_loop(0, DROWS, zden, 0)
        for t in range(DROWS // 16):
            idx_v[pl.ds(t * 16, 16)] = iota16 + (t * 16)

        @pl.when(sid == 0)
        def _():
            pltpu.sync_copy(den_v, den_sh)

        pltpu.sync_copy(src_hbm.at[sid], src_v)
        pltpu.sync_copy(dst_hbm.at[sid], dst_v)
        pltpu.sync_copy(s1_hbm, s1_v)
        pltpu.sync_copy(s2_hbm, s2_v)

        plsc.subcore_barrier()

        def chunk_body(g, _):
            gather = pltpu.async_copy(
                h_hbm.at[cid].at[src_v.at[g]], rows_v, sem)
            for j in range(CHUNK // 16):
                sv = src_v[g, pl.ds(j * 16, 16)]
                dv = dst_v[g, pl.ds(j * 16, 16)]
                e = plsc.load_gather(s1_v, [sv]) + plsc.load_gather(s2_v, [dv])
                e = jnp.where(e > 0.0, e, 0.2 * e)
                w = jnp.exp(e)
                w_v[pl.ds(j * 16, 16)] = w
                plsc.addupdate_scatter(
                    den_v, [lax.shift_right_logical(dv, 7),
                            lax.bitwise_and(dv, 127)], w)
            gather.wait()

            def scale_body(r, _):
                wv = plsc.load_gather(w_v, [jnp.full((16,), r, jnp.int32)])
                for cc in range(HFEAT // 16):
                    scaled_v[r, pl.ds(cc * 16, 16)] = (
                        rows_v[r, pl.ds(cc * 16, 16)] * wv)
                return 0

            lax.fori_loop(0, CHUNK, scale_body, 0)
            pltpu.sync_copy(scaled_v, acc_sh.at[dst_v.at[g]], add=True)
            return 0

        lax.fori_loop(0, NCHUNK, chunk_body, 0)

        pltpu.sync_copy(den_v, den_sh.at[idx_v], add=True)
        plsc.subcore_barrier()

        pltpu.sync_copy(den_sh.at[pl.ds(sid * DRPT, DRPT)],
                        den_hbm.at[cid, pl.ds(sid * DRPT, DRPT)])
        pltpu.sync_copy(acc_sh.at[pl.ds(sid * RPT, RPT)],
                        out_hbm.at[cid, pl.ds(sid * RPT, RPT)])

    return k(hsplit, s1, s2, src3, dst3)


def _finalize_body(p_ref, d_ref, o_ref):
    d0 = d_ref[0]
    d1 = d_ref[1]
    d0 = jnp.where(d0 == 0.0, 1.0, d0)
    d1 = jnp.where(d1 == 0.0, 1.0, d1)
    x = jnp.concatenate([p_ref[0] / d0, p_ref[1] / d1], axis=1)
    o_ref[...] = jnp.where(x > 0.0, x, jnp.exp(jnp.minimum(x, 0.0)) - 1.0)


def _finalize(partials, dens):
    blk = 2048
    return pl.pallas_call(
        _finalize_body,
        grid=(N_PAD // blk,),
        in_specs=[
            pl.BlockSpec((NC, blk, HFEAT), lambda i: (0, i, 0)),
            pl.BlockSpec((NC, blk, 1), lambda i: (0, i, 0)),
        ],
        out_specs=pl.BlockSpec((blk, FEAT), lambda i: (i, 0)),
        out_shape=jax.ShapeDtypeStruct((N_PAD, FEAT), jnp.float32),
    )(partials, dens)


def kernel(features, edge_src, edge_dst, W, a):
    x = features.reshape(N_NODES, FEAT)
    a2 = a.reshape(2, FEAT).T
    hsplit, s = _project(x, W, a2)
    s1 = s[:, 0]
    s2 = s[:, 1]
    src3 = edge_src.reshape(NS, NCHUNK, CHUNK)
    dst3 = edge_dst.reshape(NS, NCHUNK, CHUNK)
    partials, dens = _sc_edge_pass(hsplit, s1, s2, src3, dst3)
    out = _finalize(partials, dens.reshape(NC, N_PAD, 1))
    return out[:N_NODES].reshape(N_NODES, 1, FEAT)
